# baseline (speedup 1.0000x reference)
import jax
import jax.numpy as jnp
from jax.experimental import pallas as pl


def _gat_conv(x, edge_index, W, a_src, a_dst, bias, heads, out_ch):
    n = x.shape[0]
    src = edge_index[0]
    dst = edge_index[1]
    h = (x @ W).reshape(n, heads, out_ch)
    alpha_src = jnp.sum(h * a_src, axis=-1)
    alpha_dst = jnp.sum(h * a_dst, axis=-1)
    e = alpha_src[src] + alpha_dst[dst]
    e = jnp.where(e > 0, e, 0.2 * e)
    m = jax.ops.segment_max(e, dst, num_segments=n)
    m = jnp.where(jnp.isfinite(m), m, 0.0)
    ex = jnp.exp(e - m[dst])
    den = jax.ops.segment_sum(ex, dst, num_segments=n)
    alpha = ex / (den[dst] + 1e-16)
    msg = h[src] * alpha[:, :, None]
    out = jax.ops.segment_sum(msg, dst, num_segments=n)
    return out.reshape(n, heads * out_ch) + bias


def _logsoftmax_kernel(h_ref, o_ref):
    h = h_ref[...]
    lane = jax.lax.broadcasted_iota(jnp.int32, h.shape, 1)
    valid = lane < 6
    hm = jnp.where(valid, h, -jnp.inf)
    m = jnp.max(hm, axis=1, keepdims=True)
    s = jnp.sum(jnp.where(valid, jnp.exp(h - m), 0.0), axis=1, keepdims=True)
    o_ref[...] = h - m - jnp.log(s)


def kernel(x, edge_index, W1, a_s1, a_d1, b1, W2, a_s2, a_d2, b2):
    h = _gat_conv(x, edge_index, W1, a_s1, a_d1, b1, heads=4, out_ch=16)
    h = jax.nn.elu(h)
    h = _gat_conv(h, edge_index, W2, a_s2, a_d2, b2, heads=1, out_ch=6)
    n = h.shape[0]
    npad = (n + 1023) // 1024 * 1024
    hp = jnp.pad(h, ((0, npad - n), (0, 122)))
    out = pl.pallas_call(
        _logsoftmax_kernel,
        out_shape=jax.ShapeDtypeStruct((npad, 128), jnp.float32),
        grid=(npad // 1024,),
        in_specs=[pl.BlockSpec((1024, 128), lambda i: (i, 0))],
        out_specs=pl.BlockSpec((1024, 128), lambda i: (i, 0)),
    )(hp)
    return out[:n, :6]


# speedup vs baseline: 33.8299x; 33.8299x over previous
import dataclasses
import functools

import numpy as np

import jax
import jax.numpy as jnp
from jax import lax
from jax.experimental import pallas as pl
from jax.experimental.pallas import tpu as pltpu
from jax.experimental.pallas import tpu_sc as plsc

N = 100000
E = 1600000
NC = 2
NS = 16
NW = NC * NS

NBINS = 8
CHUNK = 12544
CAP = 50176
FLAT = NW * NBINS * CAP
BINSZ = FLAT + NW * 64

WIN = 2000
NWINDOWS = E // WIN
WPT = NWINDOWS // NW

NPAD = 100352
ACC1_ROWS = 12672
HALF2 = 50176
ACC2_ROWS = 50304
BAT = 256

_mesh = functools.partial(
    plsc.VectorSubcoreMesh, core_axis_name="c", subcore_axis_name="s")


def _sc_params():
    cp = pltpu.CompilerParams()
    fields = pltpu.CompilerParams.__dataclass_fields__
    if "needs_layout_passes" in fields:
        cp = dataclasses.replace(cp, needs_layout_passes=False)
    if "use_tc_tiling_on_sc" in fields:
        cp = dataclasses.replace(cp, use_tc_tiling_on_sc=False)
    return cp

_GATHER_DNUMS = lax.GatherDimensionNumbers(
    offset_dims=(), collapsed_slice_dims=(0,), start_index_map=(0,))


def _lanes(x, idx):
    return lax.gather(x, idx[:, None], _GATHER_DNUMS, slice_sizes=(1,),
                      mode=lax.GatherScatterMode.PROMISE_IN_BOUNDS)


def _sc_bin(src, dst):
    outs = [jax.ShapeDtypeStruct((BINSZ,), jnp.int32),
            jax.ShapeDtypeStruct((BINSZ,), jnp.int32),
            jax.ShapeDtypeStruct((NW, 16), jnp.int32)]
    scratch = [pltpu.VMEM((2048,), jnp.int32),
               pltpu.VMEM((2048,), jnp.int32),
               pltpu.VMEM((16, 128), jnp.int32),
               pltpu.VMEM((16,), jnp.int32),
               pltpu.SemaphoreType.DMA,
               pltpu.SemaphoreType.DMA]

    @functools.partial(pl.kernel, mesh=_mesh(), out_type=outs,
                       scratch_types=scratch, compiler_params=_sc_params())
    def k(src_h, dst_h, bs_h, bd_h, cnt_h, ebs, ebd, offs, cntbuf, sm1, sm2):
        wid = lax.axis_index("s") * NC + lax.axis_index("c")
        iota = lax.iota(jnp.int32, 16)
        dumpbase = FLAT + wid * 64
        curs0 = tuple(
            jnp.full((16,), (wid * NBINS + q) * CAP, jnp.int32)
            for q in range(NBINS))

        def win_body(wi, curs):
            off_e = (wid + NW * wi) * WIN
            pltpu.sync_copy(src_h.at[pl.ds(off_e, WIN)], ebs.at[pl.ds(0, WIN)])
            pltpu.sync_copy(dst_h.at[pl.ds(off_e, WIN)], ebd.at[pl.ds(0, WIN)])
            for r in range(16):
                nvr = 8 if r < 15 else 5

                def vbody(jj, curs, r=r, nvr=nvr):
                    j = r * 8 + jj
                    dv = ebd[pl.ds(j * 16, 16)]
                    cb = dv // CHUNK
                    offv = jnp.zeros((16,), jnp.int32)
                    newcurs = []
                    for q in range(NBINS):
                        mq = cb == q
                        mqi = mq.astype(jnp.int32)
                        rank = plsc.cumsum(mqi) - mqi
                        cntq = plsc.all_reduce_population_count(mq)
                        offv = jnp.where(mq, curs[q] + rank, offv)
                        newcurs.append(curs[q] + cntq)
                    offs[r, pl.ds(jj * 16, 16)] = offv
                    return tuple(newcurs)

                curs = lax.fori_loop(0, nvr, vbody, curs)
            for tpos in range(3):
                offs[15, pl.ds(80 + tpos * 16, 16)] = (
                    dumpbase + tpos * 16 + iota)
            cps = []
            for j2 in range(16):
                cps.append(pltpu.async_copy(
                    ebs.at[pl.ds(j2 * 128, 128)], bs_h.at[offs.at[j2]], sm1))
                cps.append(pltpu.async_copy(
                    ebd.at[pl.ds(j2 * 128, 128)], bd_h.at[offs.at[j2]], sm2))
            for cp in cps:
                cp.wait()
            return curs

        curs = lax.fori_loop(0, WPT, win_body, curs0)
        cv = jnp.zeros((16,), jnp.int32)
        for q in range(NBINS):
            cv = jnp.where(iota == q, curs[q], cv)
        cntbuf[...] = cv
        pltpu.sync_copy(cntbuf, cnt_h.at[wid])

    return k(src, dst)


def _sc_l1(bs, bd, cnts, htab, atab):
    out = jax.ShapeDtypeStruct((NPAD, 80), jnp.float32)
    scratch = [pltpu.VMEM_SHARED((ACC1_ROWS, 80), jnp.float32),
               pltpu.VMEM((BAT,), jnp.int32),
               pltpu.VMEM((BAT,), jnp.int32),
               pltpu.VMEM((2, 128), jnp.int32),
               pltpu.VMEM((BAT, 64), jnp.float32),
               pltpu.VMEM((BAT, 16), jnp.float32),
               pltpu.VMEM((BAT, 16), jnp.float32),
               pltpu.VMEM((BAT, 80), jnp.float32),
               pltpu.VMEM((NW, 16), jnp.int32),
               pltpu.SemaphoreType.DMA,
               pltpu.SemaphoreType.DMA,
               pltpu.SemaphoreType.DMA]

    @functools.partial(pl.kernel, mesh=_mesh(), out_type=out,
                       scratch_types=scratch, compiler_params=_sc_params())
    def k(bs_h, bd_h, cnt_h, ht_h, at_h, out_h, acc, srcb, dstb, dloc,
          hrows, asr, adr, staged, cnt_v, s1, s2, s3):
        core = lax.axis_index("c")
        sid = lax.axis_index("s")
        iota = lax.iota(jnp.int32, 16)
        perm = jnp.minimum(iota + 4, 15)
        pltpu.sync_copy(cnt_h, cnt_v)

        for k2 in range(NBINS // NC):
            q = core * (NBINS // NC) + k2
            lo = q * CHUNK

            def zb(i, c):
                for kk in range(5):
                    staged[i, pl.ds(kk * 16, 16)] = jnp.zeros(
                        (16,), jnp.float32)
                return c
            lax.fori_loop(0, 128, zb, 0)

            def zc(b, c):
                @pl.when(b % 16 == sid)
                def _():
                    pltpu.sync_copy(staged.at[pl.ds(0, 128)],
                                    acc.at[pl.ds(b * 128, 128)])
                return c
            lax.fori_loop(0, ACC1_ROWS // 128, zc, 0)
            plsc.subcore_barrier()

            for t2 in range(2):
                t = sid + t2 * 16
                base = (t * NBINS + q) * CAP
                cv = cnt_v[t, pl.ds(0, 16)]
                cnt = jnp.sum(jnp.where(iota == q, cv, 0)) - base
                nbat = (cnt + (BAT - 1)) // BAT

                def bat(b, c):
                    off = base + b * BAT
                    pltpu.sync_copy(bs_h.at[pl.ds(off, BAT)], srcb)
                    pltpu.sync_copy(bd_h.at[pl.ds(off, BAT)], dstb)
                    valid = cnt - b * BAT
                    for j in range(BAT // 16):
                        sv = srcb[pl.ds(j * 16, 16)]
                        dv = dstb[pl.ds(j * 16, 16)]
                        gid = j * 16 + iota
                        mval = gid < valid
                        safe = sid * 128 + iota
                        srcb[pl.ds(j * 16, 16)] = jnp.where(mval, sv, safe)
                        dstb[pl.ds(j * 16, 16)] = jnp.where(mval, dv, safe)
                        dloc[j // 8, pl.ds((j % 8) * 16, 16)] = jnp.where(
                            mval, dv - lo, CHUNK + iota)
                    c1 = pltpu.async_copy(ht_h.at[srcb], hrows, s1)
                    c2 = pltpu.async_copy(at_h.at[srcb], asr, s2)
                    c3 = pltpu.async_copy(at_h.at[dstb], adr, s3)
                    c1.wait()
                    c2.wait()
                    c3.wait()

                    def edge(i, c2_):
                        av = asr[i, pl.ds(0, 16)]
                        bv = adr[i, pl.ds(0, 16)]
                        bvp = _lanes(bv, perm)
                        e = av + bvp
                        w = jnp.exp(jnp.maximum(e, 0.2 * e))
                        staged[i, pl.ds(64, 16)] = w
                        for h in range(4):
                            wb = _lanes(w, jnp.full((16,), h, jnp.int32))
                            staged[i, pl.ds(h * 16, 16)] = (
                                hrows[i, pl.ds(h * 16, 16)] * wb)
                        return c2_
                    lax.fori_loop(0, BAT, edge, 0)
                    for j2 in range(BAT // 128):
                        pltpu.sync_copy(
                            staged.at[pl.ds(j2 * 128, 128)],
                            acc.at[dloc.at[j2]], add=True)
                    return c
                lax.fori_loop(0, nbat, bat, 0)
            plsc.subcore_barrier()

            def co(r, c):
                blk = sid + 16 * r
                pltpu.sync_copy(acc.at[pl.ds(blk * 196, 196)],
                                out_h.at[pl.ds(lo + blk * 196, 196)])
                return c
            lax.fori_loop(0, 4, co, 0)
            plsc.subcore_barrier()

    return k(bs, bd, cnts, htab, atab)


def _sc_l2(bs, bd, cnts, h2tab):
    out = jax.ShapeDtypeStruct((NPAD, 16), jnp.float32)
    scratch = [pltpu.VMEM_SHARED((ACC2_ROWS, 16), jnp.float32),
               pltpu.VMEM((BAT,), jnp.int32),
               pltpu.VMEM((BAT,), jnp.int32),
               pltpu.VMEM((2, 128), jnp.int32),
               pltpu.VMEM((BAT, 16), jnp.float32),
               pltpu.VMEM((BAT, 16), jnp.float32),
               pltpu.VMEM((BAT, 16), jnp.float32),
               pltpu.VMEM((NW, 16), jnp.int32),
               pltpu.SemaphoreType.DMA,
               pltpu.SemaphoreType.DMA]

    @functools.partial(pl.kernel, mesh=_mesh(), out_type=out,
                       scratch_types=scratch, compiler_params=_sc_params())
    def k(bs_h, bd_h, cnt_h, ht_h, out_h, acc, srcb, dstb, dloc, hr, hd,
          staged, cnt_v, s1, s2):
        core = lax.axis_index("c")
        sid = lax.axis_index("s")
        iota = lax.iota(jnp.int32, 16)
        sp6 = jnp.full((16,), 6, jnp.int32)
        sp7 = jnp.full((16,), 7, jnp.int32)
        sel6 = (iota < 6).astype(jnp.float32)
        oh6 = (iota == 6).astype(jnp.float32)
        pltpu.sync_copy(cnt_h, cnt_v)
        lo = core * HALF2

        def zb(i, c):
            staged[i, pl.ds(0, 16)] = jnp.zeros((16,), jnp.float32)
            return c
        lax.fori_loop(0, 128, zb, 0)

        def zc(b, c):
            @pl.when(b % 16 == sid)
            def _():
                pltpu.sync_copy(staged.at[pl.ds(0, 128)],
                                acc.at[pl.ds(b * 128, 128)])
            return c
        lax.fori_loop(0, ACC2_ROWS // 128, zc, 0)
        plsc.subcore_barrier()

        for k2 in range(NBINS // NC):
            q = core * (NBINS // NC) + k2
            for t2 in range(2):
                t = sid + t2 * 16
                base = (t * NBINS + q) * CAP
                cv = cnt_v[t, pl.ds(0, 16)]
                cnt = jnp.sum(jnp.where(iota == q, cv, 0)) - base
                nbat = (cnt + (BAT - 1)) // BAT

                def bat(b, c):
                    off = base + b * BAT
                    pltpu.sync_copy(bs_h.at[pl.ds(off, BAT)], srcb)
                    pltpu.sync_copy(bd_h.at[pl.ds(off, BAT)], dstb)
                    valid = cnt - b * BAT
                    for j in range(BAT // 16):
                        sv = srcb[pl.ds(j * 16, 16)]
                        dv = dstb[pl.ds(j * 16, 16)]
                        gid = j * 16 + iota
                        mval = gid < valid
                        safe = sid * 128 + iota
                        srcb[pl.ds(j * 16, 16)] = jnp.where(mval, sv, safe)
                        dstb[pl.ds(j * 16, 16)] = jnp.where(mval, dv, safe)
                        dloc[j // 8, pl.ds((j % 8) * 16, 16)] = jnp.where(
                            mval, dv - lo, HALF2 + iota)
                    c1 = pltpu.async_copy(ht_h.at[srcb], hr, s1)
                    c2 = pltpu.async_copy(ht_h.at[dstb], hd, s2)
                    c1.wait()
                    c2.wait()

                    def edge(i, c2_):
                        av = hr[i, pl.ds(0, 16)]
                        bv = hd[i, pl.ds(0, 16)]
                        ea = _lanes(av, sp6) + _lanes(bv, sp7)
                        w = jnp.exp(jnp.maximum(ea, 0.2 * ea))
                        staged[i, pl.ds(0, 16)] = (av * sel6 + oh6) * w
                        return c2_
                    lax.fori_loop(0, BAT, edge, 0)
                    for j2 in range(BAT // 128):
                        pltpu.sync_copy(
                            staged.at[pl.ds(j2 * 128, 128)],
                            acc.at[dloc.at[j2]], add=True)
                    return c
                lax.fori_loop(0, nbat, bat, 0)
        plsc.subcore_barrier()

        def co(r, c):
            blk = sid + 16 * r
            pltpu.sync_copy(acc.at[pl.ds(blk * 224, 224)],
                            out_h.at[pl.ds(lo + blk * 224, 224)])
            return c
        lax.fori_loop(0, 14, co, 0)

    return k(bs, bd, cnts, h2tab)


def _tc1(xp, wc1):
    def body(x_ref, w_ref, h_ref, a_ref):
        xb = x_ref[...]
        h_ref[...] = jnp.dot(xb, w_ref[:, :64],
                             preferred_element_type=jnp.float32)
        a_ref[...] = jnp.dot(xb, w_ref[:, 64:80],
                             preferred_element_type=jnp.float32)

    return pl.pallas_call(
        body,
        grid=(NPAD // 512,),
        in_specs=[pl.BlockSpec((512, 8), lambda i: (i, 0)),
                  pl.BlockSpec((8, 80), lambda i: (0, 0))],
        out_specs=[pl.BlockSpec((512, 64), lambda i: (i, 0)),
                   pl.BlockSpec((512, 16), lambda i: (i, 0))],
        out_shape=[jax.ShapeDtypeStruct((NPAD, 64), jnp.float32),
                   jax.ShapeDtypeStruct((NPAD, 16), jnp.float32)],
    )(xp, wc1)


def _tc2(acc1, r4, b1, wc2):
    def body(a_ref, r_ref, b_ref, w_ref, o_ref):
        a = a_ref[...]
        den = jnp.dot(a[:, 64:68], r_ref[...],
                      preferred_element_type=jnp.float32)
        o1 = a[:, :64] / (den + 1e-16) + b_ref[...]
        o1 = jnp.where(o1 > 0, o1, jnp.exp(o1) - 1.0)
        o_ref[...] = jnp.dot(o1, w_ref[...],
                             preferred_element_type=jnp.float32)

    return pl.pallas_call(
        body,
        grid=(NPAD // 512,),
        in_specs=[pl.BlockSpec((512, 80), lambda i: (i, 0)),
                  pl.BlockSpec((4, 64), lambda i: (0, 0)),
                  pl.BlockSpec((1, 64), lambda i: (0, 0)),
                  pl.BlockSpec((64, 16), lambda i: (0, 0))],
        out_specs=pl.BlockSpec((512, 16), lambda i: (i, 0)),
        out_shape=jax.ShapeDtypeStruct((NPAD, 16), jnp.float32),
    )(acc1, r4, b1, wc2)


def _tc3(acc2, seln, seld, b2p):
    def body(a_ref, sn_ref, sd_ref, b_ref, o_ref):
        a = a_ref[...]
        num = jnp.dot(a, sn_ref[...], preferred_element_type=jnp.float32)
        den = jnp.dot(a, sd_ref[...], preferred_element_type=jnp.float32)
        h = num / (den + 1e-16) + b_ref[...]
        lane = lax.broadcasted_iota(jnp.int32, h.shape, 1)
        validm = lane < 6
        hm = jnp.where(validm, h, -jnp.inf)
        m = jnp.max(hm, axis=1, keepdims=True)
        s = jnp.sum(jnp.where(validm, jnp.exp(h - m), 0.0), axis=1,
                    keepdims=True)
        o_ref[...] = h - m - jnp.log(s)

    return pl.pallas_call(
        body,
        grid=(NPAD // 1024,),
        in_specs=[pl.BlockSpec((1024, 16), lambda i: (i, 0)),
                  pl.BlockSpec((16, 16), lambda i: (0, 0)),
                  pl.BlockSpec((16, 16), lambda i: (0, 0)),
                  pl.BlockSpec((1, 16), lambda i: (0, 0))],
        out_specs=pl.BlockSpec((1024, 16), lambda i: (i, 0)),
        out_shape=jax.ShapeDtypeStruct((NPAD, 16), jnp.float32),
    )(acc2, seln, seld, b2p)


def kernel(x, edge_index, W1, a_s1, a_d1, b1, W2, a_s2, a_d2, b2):
    f32 = jnp.float32
    W1p = jnp.pad(W1, ((0, 1), (0, 0)))
    asf1 = a_s1.reshape(64)
    adf1 = a_d1.reshape(64)
    s4 = np.zeros((64, 16), np.float32)
    for jj in range(64):
        s4[jj, jj // 16] = 1.0
    s4b = np.zeros((64, 16), np.float32)
    for jj in range(64):
        s4b[jj, 4 + jj // 16] = 1.0
    amat = W1p @ (asf1[:, None] * jnp.asarray(s4)) \
        + W1p @ (adf1[:, None] * jnp.asarray(s4b))
    wc1 = jnp.concatenate([W1p, amat], axis=1)

    r4 = np.zeros((4, 64), np.float32)
    for hh in range(4):
        r4[hh, hh * 16:(hh + 1) * 16] = 1.0
    as2v = a_s2.reshape(6)
    ad2v = a_d2.reshape(6)
    wc2 = jnp.concatenate(
        [W2, (W2 @ as2v)[:, None], (W2 @ ad2v)[:, None],
         jnp.zeros((64, 8), f32)], axis=1)

    seln = np.zeros((16, 16), np.float32)
    for jj in range(6):
        seln[jj, jj] = 1.0
    seld = np.zeros((16, 16), np.float32)
    seld[6, 0:6] = 1.0
    b2p = jnp.pad(b2, (0, 10)).reshape(1, 16)

    xp = jnp.pad(x, ((0, NPAD - N), (0, 1)))

    htab, atab = _tc1(xp, wc1)
    src = edge_index[0]
    dst = edge_index[1]
    bs, bd, cnts = _sc_bin(src, dst)
    acc1 = _sc_l1(bs, bd, cnts, htab, atab)
    h2tab = _tc2(acc1, jnp.asarray(r4), b1.reshape(1, 64), wc2)
    acc2 = _sc_l2(bs, bd, cnts, h2tab)
    out = _tc3(acc2, jnp.asarray(seln), jnp.asarray(seld), b2p)
    return out[:N, :6]


# speedup vs baseline: 64.3719x; 1.9028x over previous
import dataclasses
import functools

import numpy as np

import jax
import jax.numpy as jnp
from jax import lax
from jax.experimental import pallas as pl
from jax.experimental.pallas import tpu as pltpu
from jax.experimental.pallas import tpu_sc as plsc

N = 100000
E = 1600000
NC = 2
NS = 16
NW = NC * NS

NBINS = 8
CHUNK = 12544
CAP = 50176
FLAT = NW * NBINS * CAP
BINSZ = FLAT + NW * 64

WIN = 2000
NWINDOWS = E // WIN
WPT = NWINDOWS // NW

NPAD = 100352
ACC1_ROWS = 12672
HALF2 = 50176
ACC2_ROWS = 50304
BAT = 256

_mesh = functools.partial(
    plsc.VectorSubcoreMesh, core_axis_name="c", subcore_axis_name="s")


def _sc_params():
    cp = pltpu.CompilerParams()
    fields = pltpu.CompilerParams.__dataclass_fields__
    if "needs_layout_passes" in fields:
        cp = dataclasses.replace(cp, needs_layout_passes=False)
    if "use_tc_tiling_on_sc" in fields:
        cp = dataclasses.replace(cp, use_tc_tiling_on_sc=False)
    return cp

_GATHER_DNUMS = lax.GatherDimensionNumbers(
    offset_dims=(), collapsed_slice_dims=(0,), start_index_map=(0,))


def _lanes(x, idx):
    return lax.gather(x, idx[:, None], _GATHER_DNUMS, slice_sizes=(1,),
                      mode=lax.GatherScatterMode.PROMISE_IN_BOUNDS)


def _sc_bin(src, dst):
    outs = [jax.ShapeDtypeStruct((BINSZ,), jnp.int32),
            jax.ShapeDtypeStruct((BINSZ,), jnp.int32),
            jax.ShapeDtypeStruct((NW, 16), jnp.int32)]
    scratch = [pltpu.VMEM((2048,), jnp.int32),
               pltpu.VMEM((2048,), jnp.int32),
               pltpu.VMEM((NBINS, 1040), jnp.int32),
               pltpu.VMEM((NBINS, 1040), jnp.int32),
               pltpu.VMEM((16,), jnp.int32)]

    @functools.partial(pl.kernel, mesh=_mesh(), out_type=outs,
                       scratch_types=scratch, compiler_params=_sc_params())
    def k(src_h, dst_h, bs_h, bd_h, cnt_h, ebs, ebd, ring_s, ring_d, cntbuf):
        wid = lax.axis_index("s") * NC + lax.axis_index("c")
        iota = lax.iota(jnp.int32, 16)
        bases = [None] * NBINS
        st0 = []
        for q in range(NBINS):
            bases[q] = (wid * NBINS + q) * CAP
            st0 += [jnp.int32(0), bases[q] + 0]
        st0 = tuple(st0)

        def win_body(wi, st):
            off_e = pl.multiple_of((wid + NW * wi) * WIN, 16)
            pltpu.sync_copy(src_h.at[pl.ds(off_e, WIN)], ebs.at[pl.ds(0, WIN)])
            pltpu.sync_copy(dst_h.at[pl.ds(off_e, WIN)], ebd.at[pl.ds(0, WIN)])

            def vbody(j, st):
                sv = ebs[pl.ds(j * 16, 16)]
                dv = ebd[pl.ds(j * 16, 16)]
                cb = dv // CHUNK
                nst = []
                for q in range(NBINS):
                    pq, wq = st[2 * q], st[2 * q + 1]
                    mq = cb == q
                    plsc.store_compressed(ring_s.at[q, pl.ds(pq, 16)], sv, mask=mq)
                    plsc.store_compressed(ring_d.at[q, pl.ds(pq, 16)], dv, mask=mq)
                    p2 = pq + jnp.sum(mq.astype(jnp.int32))
                    cross_a = jnp.logical_and(pq < 512, p2 >= 512)
                    cross_b = p2 >= 1024

                    @pl.when(cross_a)
                    def _(q=q, wq=wq):
                        wqa = pl.multiple_of(wq, 512)
                        pltpu.sync_copy(ring_s.at[q, pl.ds(0, 512)],
                                        bs_h.at[pl.ds(wqa, 512)])
                        pltpu.sync_copy(ring_d.at[q, pl.ds(0, 512)],
                                        bd_h.at[pl.ds(wqa, 512)])

                    @pl.when(cross_b)
                    def _(q=q, wq=wq):
                        wqa = pl.multiple_of(wq, 512)
                        pltpu.sync_copy(ring_s.at[q, pl.ds(512, 512)],
                                        bs_h.at[pl.ds(wqa, 512)])
                        pltpu.sync_copy(ring_d.at[q, pl.ds(512, 512)],
                                        bd_h.at[pl.ds(wqa, 512)])
                        ring_s[q, pl.ds(0, 16)] = ring_s[q, pl.ds(1024, 16)]
                        ring_d[q, pl.ds(0, 16)] = ring_d[q, pl.ds(1024, 16)]

                    crossed = jnp.logical_or(cross_a, cross_b)
                    wq = jnp.where(crossed, wq + 512, wq)
                    p2 = jnp.where(cross_b, p2 - 1024, p2)
                    nst += [p2, wq]
                return tuple(nst)

            return lax.fori_loop(0, WIN // 16, vbody, st)

        st = lax.fori_loop(0, WPT, win_body, st0)
        cv = jnp.zeros((16,), jnp.int32)
        for q in range(NBINS):
            pq, wq = st[2 * q], st[2 * q + 1]
            cs = ((wq - bases[q]) >> 9 & 1) * 512

            @pl.when(pq > cs)
            def _(q=q, pq=pq, wq=wq, cs=cs):
                wqa = pl.multiple_of(wq, 512)
                pltpu.sync_copy(ring_s.at[q, pl.ds(cs, 512)],
                                bs_h.at[pl.ds(wqa, 512)])
                pltpu.sync_copy(ring_d.at[q, pl.ds(cs, 512)],
                                bd_h.at[pl.ds(wqa, 512)])

            cv = jnp.where(iota == q, wq + pq - cs, cv)
        cntbuf[...] = cv
        pltpu.sync_copy(cntbuf, cnt_h.at[wid])

    return k(src, dst)


def _sc_l1(bs, bd, cnts, htab, atab):
    out = jax.ShapeDtypeStruct((NPAD, 80), jnp.float32)
    scratch = [pltpu.VMEM_SHARED((ACC1_ROWS, 80), jnp.float32),
               pltpu.VMEM((BAT,), jnp.int32),
               pltpu.VMEM((BAT,), jnp.int32),
               pltpu.VMEM((2, 128), jnp.int32),
               pltpu.VMEM((BAT, 64), jnp.float32),
               pltpu.VMEM((BAT, 16), jnp.float32),
               pltpu.VMEM((BAT, 16), jnp.float32),
               pltpu.VMEM((BAT, 80), jnp.float32),
               pltpu.VMEM((NW, 16), jnp.int32),
               pltpu.SemaphoreType.DMA,
               pltpu.SemaphoreType.DMA,
               pltpu.SemaphoreType.DMA]

    @functools.partial(pl.kernel, mesh=_mesh(), out_type=out,
                       scratch_types=scratch, compiler_params=_sc_params())
    def k(bs_h, bd_h, cnt_h, ht_h, at_h, out_h, acc, srcb, dstb, dloc,
          hrows, asr, adr, staged, cnt_v, s1, s2, s3):
        core = lax.axis_index("c")
        sid = lax.axis_index("s")
        iota = lax.iota(jnp.int32, 16)
        perm = jnp.minimum(iota + 4, 15)
        pltpu.sync_copy(cnt_h, cnt_v)

        for k2 in range(NBINS // NC):
            q = core * (NBINS // NC) + k2
            lo = q * CHUNK

            def zb(i, c):
                for kk in range(5):
                    staged[i, pl.ds(kk * 16, 16)] = jnp.zeros(
                        (16,), jnp.float32)
                return c
            lax.fori_loop(0, 128, zb, 0)

            def zc(b, c):
                @pl.when(b % 16 == sid)
                def _():
                    pltpu.sync_copy(staged.at[pl.ds(0, 128)],
                                    acc.at[pl.ds(b * 128, 128)])
                return c
            lax.fori_loop(0, ACC1_ROWS // 128, zc, 0)
            plsc.subcore_barrier()

            for t2 in range(2):
                t = sid + t2 * 16
                base = (t * NBINS + q) * CAP
                cv = cnt_v[t, pl.ds(0, 16)]
                cnt = jnp.sum(jnp.where(iota == q, cv, 0)) - base
                nbat = (cnt + (BAT - 1)) // BAT

                def bat(b, c):
                    off = pl.multiple_of(base + b * BAT, 256)
                    pltpu.sync_copy(bs_h.at[pl.ds(off, BAT)], srcb)
                    pltpu.sync_copy(bd_h.at[pl.ds(off, BAT)], dstb)
                    valid = cnt - b * BAT
                    for j in range(BAT // 16):
                        sv = srcb[pl.ds(j * 16, 16)]
                        dv = dstb[pl.ds(j * 16, 16)]
                        gid = j * 16 + iota
                        mval = gid < valid
                        safe = sid * 128 + iota
                        srcb[pl.ds(j * 16, 16)] = jnp.where(mval, sv, safe)
                        dstb[pl.ds(j * 16, 16)] = jnp.where(mval, dv, safe)
                        dloc[j // 8, pl.ds((j % 8) * 16, 16)] = jnp.where(
                            mval, dv - lo, CHUNK + iota)
                    c1 = pltpu.async_copy(ht_h.at[srcb], hrows, s1)
                    c2 = pltpu.async_copy(at_h.at[srcb], asr, s2)
                    c3 = pltpu.async_copy(at_h.at[dstb], adr, s3)
                    c1.wait()
                    c2.wait()
                    c3.wait()

                    def edge(i, c2_):
                        av = asr[i, pl.ds(0, 16)]
                        bv = adr[i, pl.ds(0, 16)]
                        bvp = _lanes(bv, perm)
                        e = av + bvp
                        w = jnp.exp(jnp.maximum(e, 0.2 * e))
                        staged[i, pl.ds(64, 16)] = w
                        for h in range(4):
                            wb = _lanes(w, jnp.full((16,), h, jnp.int32))
                            staged[i, pl.ds(h * 16, 16)] = (
                                hrows[i, pl.ds(h * 16, 16)] * wb)
                        return c2_
                    lax.fori_loop(0, BAT, edge, 0)
                    for j2 in range(BAT // 128):
                        pltpu.sync_copy(
                            staged.at[pl.ds(j2 * 128, 128)],
                            acc.at[dloc.at[j2]], add=True)
                    return c
                lax.fori_loop(0, nbat, bat, 0)
            plsc.subcore_barrier()

            def co(r, c):
                blk = sid + 16 * r
                pltpu.sync_copy(acc.at[pl.ds(blk * 196, 196)],
                                out_h.at[pl.ds(lo + blk * 196, 196)])
                return c
            lax.fori_loop(0, 4, co, 0)
            plsc.subcore_barrier()

    return k(bs, bd, cnts, htab, atab)


def _sc_l2(bs, bd, cnts, h2tab):
    out = jax.ShapeDtypeStruct((NPAD, 16), jnp.float32)
    scratch = [pltpu.VMEM_SHARED((ACC2_ROWS, 16), jnp.float32),
               pltpu.VMEM((BAT,), jnp.int32),
               pltpu.VMEM((BAT,), jnp.int32),
               pltpu.VMEM((2, 128), jnp.int32),
               pltpu.VMEM((BAT, 16), jnp.float32),
               pltpu.VMEM((BAT, 16), jnp.float32),
               pltpu.VMEM((BAT, 16), jnp.float32),
               pltpu.VMEM((NW, 16), jnp.int32),
               pltpu.SemaphoreType.DMA,
               pltpu.SemaphoreType.DMA]

    @functools.partial(pl.kernel, mesh=_mesh(), out_type=out,
                       scratch_types=scratch, compiler_params=_sc_params())
    def k(bs_h, bd_h, cnt_h, ht_h, out_h, acc, srcb, dstb, dloc, hr, hd,
          staged, cnt_v, s1, s2):
        core = lax.axis_index("c")
        sid = lax.axis_index("s")
        iota = lax.iota(jnp.int32, 16)
        sp6 = jnp.full((16,), 6, jnp.int32)
        sp7 = jnp.full((16,), 7, jnp.int32)
        sel6 = (iota < 6).astype(jnp.float32)
        oh6 = (iota == 6).astype(jnp.float32)
        pltpu.sync_copy(cnt_h, cnt_v)
        lo = core * HALF2

        def zb(i, c):
            staged[i, pl.ds(0, 16)] = jnp.zeros((16,), jnp.float32)
            return c
        lax.fori_loop(0, 128, zb, 0)

        def zc(b, c):
            @pl.when(b % 16 == sid)
            def _():
                pltpu.sync_copy(staged.at[pl.ds(0, 128)],
                                acc.at[pl.ds(b * 128, 128)])
            return c
        lax.fori_loop(0, ACC2_ROWS // 128, zc, 0)
        plsc.subcore_barrier()

        for k2 in range(NBINS // NC):
            q = core * (NBINS // NC) + k2
            for t2 in range(2):
                t = sid + t2 * 16
                base = (t * NBINS + q) * CAP
                cv = cnt_v[t, pl.ds(0, 16)]
                cnt = jnp.sum(jnp.where(iota == q, cv, 0)) - base
                nbat = (cnt + (BAT - 1)) // BAT

                def bat(b, c):
                    off = pl.multiple_of(base + b * BAT, 256)
                    pltpu.sync_copy(bs_h.at[pl.ds(off, BAT)], srcb)
                    pltpu.sync_copy(bd_h.at[pl.ds(off, BAT)], dstb)
                    valid = cnt - b * BAT
                    for j in range(BAT // 16):
                        sv = srcb[pl.ds(j * 16, 16)]
                        dv = dstb[pl.ds(j * 16, 16)]
                        gid = j * 16 + iota
                        mval = gid < valid
                        safe = sid * 128 + iota
                        srcb[pl.ds(j * 16, 16)] = jnp.where(mval, sv, safe)
                        dstb[pl.ds(j * 16, 16)] = jnp.where(mval, dv, safe)
                        dloc[j // 8, pl.ds((j % 8) * 16, 16)] = jnp.where(
                            mval, dv - lo, HALF2 + iota)
                    c1 = pltpu.async_copy(ht_h.at[srcb], hr, s1)
                    c2 = pltpu.async_copy(ht_h.at[dstb], hd, s2)
                    c1.wait()
                    c2.wait()

                    def edge(i, c2_):
                        av = hr[i, pl.ds(0, 16)]
                        bv = hd[i, pl.ds(0, 16)]
                        ea = _lanes(av, sp6) + _lanes(bv, sp7)
                        w = jnp.exp(jnp.maximum(ea, 0.2 * ea))
                        staged[i, pl.ds(0, 16)] = (av * sel6 + oh6) * w
                        return c2_
                    lax.fori_loop(0, BAT, edge, 0)
                    for j2 in range(BAT // 128):
                        pltpu.sync_copy(
                            staged.at[pl.ds(j2 * 128, 128)],
                            acc.at[dloc.at[j2]], add=True)
                    return c
                lax.fori_loop(0, nbat, bat, 0)
        plsc.subcore_barrier()

        def co(r, c):
            blk = sid + 16 * r
            pltpu.sync_copy(acc.at[pl.ds(blk * 224, 224)],
                            out_h.at[pl.ds(lo + blk * 224, 224)])
            return c
        lax.fori_loop(0, 14, co, 0)

    return k(bs, bd, cnts, h2tab)


def _tc1(xp, wc1):
    def body(x_ref, w_ref, h_ref, a_ref):
        xb = x_ref[...]
        h_ref[...] = jnp.dot(xb, w_ref[:, :64],
                             preferred_element_type=jnp.float32)
        a_ref[...] = jnp.dot(xb, w_ref[:, 64:80],
                             preferred_element_type=jnp.float32)

    return pl.pallas_call(
        body,
        grid=(NPAD // 512,),
        in_specs=[pl.BlockSpec((512, 8), lambda i: (i, 0)),
                  pl.BlockSpec((8, 80), lambda i: (0, 0))],
        out_specs=[pl.BlockSpec((512, 64), lambda i: (i, 0)),
                   pl.BlockSpec((512, 16), lambda i: (i, 0))],
        out_shape=[jax.ShapeDtypeStruct((NPAD, 64), jnp.float32),
                   jax.ShapeDtypeStruct((NPAD, 16), jnp.float32)],
    )(xp, wc1)


def _tc2(acc1, r4, b1, wc2):
    def body(a_ref, r_ref, b_ref, w_ref, o_ref):
        a = a_ref[...]
        den = jnp.dot(a[:, 64:68], r_ref[...],
                      preferred_element_type=jnp.float32)
        o1 = a[:, :64] / (den + 1e-16) + b_ref[...]
        o1 = jnp.where(o1 > 0, o1, jnp.exp(o1) - 1.0)
        o_ref[...] = jnp.dot(o1, w_ref[...],
                             preferred_element_type=jnp.float32)

    return pl.pallas_call(
        body,
        grid=(NPAD // 512,),
        in_specs=[pl.BlockSpec((512, 80), lambda i: (i, 0)),
                  pl.BlockSpec((4, 64), lambda i: (0, 0)),
                  pl.BlockSpec((1, 64), lambda i: (0, 0)),
                  pl.BlockSpec((64, 16), lambda i: (0, 0))],
        out_specs=pl.BlockSpec((512, 16), lambda i: (i, 0)),
        out_shape=jax.ShapeDtypeStruct((NPAD, 16), jnp.float32),
    )(acc1, r4, b1, wc2)


def _tc3(acc2, seln, seld, b2p):
    def body(a_ref, sn_ref, sd_ref, b_ref, o_ref):
        a = a_ref[...]
        num = jnp.dot(a, sn_ref[...], preferred_element_type=jnp.float32)
        den = jnp.dot(a, sd_ref[...], preferred_element_type=jnp.float32)
        h = num / (den + 1e-16) + b_ref[...]
        lane = lax.broadcasted_iota(jnp.int32, h.shape, 1)
        validm = lane < 6
        hm = jnp.where(validm, h, -jnp.inf)
        m = jnp.max(hm, axis=1, keepdims=True)
        s = jnp.sum(jnp.where(validm, jnp.exp(h - m), 0.0), axis=1,
                    keepdims=True)
        o_ref[...] = h - m - jnp.log(s)

    return pl.pallas_call(
        body,
        grid=(NPAD // 1024,),
        in_specs=[pl.BlockSpec((1024, 16), lambda i: (i, 0)),
                  pl.BlockSpec((16, 16), lambda i: (0, 0)),
                  pl.BlockSpec((16, 16), lambda i: (0, 0)),
                  pl.BlockSpec((1, 16), lambda i: (0, 0))],
        out_specs=pl.BlockSpec((1024, 16), lambda i: (i, 0)),
        out_shape=jax.ShapeDtypeStruct((NPAD, 16), jnp.float32),
    )(acc2, seln, seld, b2p)


def kernel(x, edge_index, W1, a_s1, a_d1, b1, W2, a_s2, a_d2, b2):
    f32 = jnp.float32
    W1p = jnp.pad(W1, ((0, 1), (0, 0)))
    asf1 = a_s1.reshape(64)
    adf1 = a_d1.reshape(64)
    s4 = np.zeros((64, 16), np.float32)
    for jj in range(64):
        s4[jj, jj // 16] = 1.0
    s4b = np.zeros((64, 16), np.float32)
    for jj in range(64):
        s4b[jj, 4 + jj // 16] = 1.0
    amat = W1p @ (asf1[:, None] * jnp.asarray(s4)) \
        + W1p @ (adf1[:, None] * jnp.asarray(s4b))
    wc1 = jnp.concatenate([W1p, amat], axis=1)

    r4 = np.zeros((4, 64), np.float32)
    for hh in range(4):
        r4[hh, hh * 16:(hh + 1) * 16] = 1.0
    as2v = a_s2.reshape(6)
    ad2v = a_d2.reshape(6)
    wc2 = jnp.concatenate(
        [W2, (W2 @ as2v)[:, None], (W2 @ ad2v)[:, None],
         jnp.zeros((64, 8), f32)], axis=1)

    seln = np.zeros((16, 16), np.float32)
    for jj in range(6):
        seln[jj, jj] = 1.0
    seld = np.zeros((16, 16), np.float32)
    seld[6, 0:6] = 1.0
    b2p = jnp.pad(b2, (0, 10)).reshape(1, 16)

    xp = jnp.pad(x, ((0, NPAD - N), (0, 1)))

    htab, atab = _tc1(xp, wc1)
    src = edge_index[0]
    dst = edge_index[1]
    bs, bd, cnts = _sc_bin(src, dst)
    acc1 = _sc_l1(bs, bd, cnts, htab, atab)
    h2tab = _tc2(acc1, jnp.asarray(r4), b1.reshape(1, 64), wc2)
    acc2 = _sc_l2(bs, bd, cnts, h2tab)
    out = _tc3(acc2, jnp.asarray(seln), jnp.asarray(seld), b2p)
    return out[:N, :6]
